# Initial kernel scaffold; baseline (speedup 1.0000x reference)
import jax
import jax.numpy as jnp
from jax.experimental import pallas as pl


def kernel(wild_x, wild_edge_index, wild_batch, mutant_x, mutant_edge_index, mutant_batch, Ww1, bw1, Ww2, bw2, Wwfc1, bwfc1, Wm1, bm1, Wm2, bm2, Wmfc1, bmfc1, mlp_W0, mlp_b0, mlp_W1, mlp_b1, mlp_Wo, mlp_bo):
    raise NotImplementedError("write your pallas kernel here")


# speedup vs baseline: 11.4937x; 11.4937x over previous
import functools

import jax
import jax.numpy as jnp
from jax import lax
from jax.experimental import pallas as pl
from jax.experimental.pallas import tpu as pltpu
from jax.experimental.pallas import tpu_sc as plsc

N = 50000
E = 800000
F = 71
OUT_DIM = 142
G = 128

FP = 72
NP = 50176
HALF = 25088
TRASH = HALF
ACC_ROWS = HALF + 8
NC, NS, L = 2, 16, 16
NW = NC * NS
CH = 96
NCH = 14
BLK = CH * NCH
NBLK = 38
EPT = BLK * NBLK
EP = EPT * NS
STRIPE = HALF // NS
PTILE = NP // NW
PSUB = 224
PSEG = 136
POFF = (0, 16, 32, 48, 56)
NEG = -3.0e38


def _mesh():
    return plsc.VectorSubcoreMesh(
        core_axis_name="c", subcore_axis_name="s", num_cores=NC,
        num_subcores=NS)


_SC_PARAMS = pltpu.CompilerParams(use_tc_tiling_on_sc=False)
_SC_PARAMS_NOLAYOUT = pltpu.CompilerParams(
    use_tc_tiling_on_sc=False, needs_layout_passes=False)


def _localize(dvm, lbuf, base):
    for k in range(BLK // L):
        dv = dvm[pl.ds(k * L, L)]
        local = dv - base
        m = (local >= 0) & (local < HALF)
        lidx = jnp.where(m, local, TRASH)
        lbuf[k // (CH // L), pl.ds((k % (CH // L)) * L, L)] = lidx


def _zero_acc(zeros_hbm, acc, s):
    pltpu.sync_copy(zeros_hbm.at[pl.ds(0, STRIPE)],
                    acc.at[pl.ds(s * STRIPE, STRIPE)])
    @pl.when(s == NS - 1)
    def _():
        pltpu.sync_copy(zeros_hbm.at[pl.ds(0, ACC_ROWS - HALF)],
                        acc.at[pl.ds(HALF, ACC_ROWS - HALF)])


def _degree_body(dst_hbm, ones_hbm, zeros_hbm, out_hbm, dvm, lbuf, ovm, acc,
                 sem):
    c = lax.axis_index("c")
    s = lax.axis_index("s")
    base = c * HALF
    _zero_acc(zeros_hbm, acc, s)
    pltpu.sync_copy(ones_hbm, ovm)
    plsc.subcore_barrier()

    def body(b, _):
        off = s * EPT + b * BLK
        pltpu.sync_copy(dst_hbm.at[pl.ds(off, BLK)], dvm)
        _localize(dvm, lbuf, base)
        descs = [pltpu.async_copy(ovm, acc.at[lbuf.at[j]], sem, add=True)
                 for j in range(NCH)]
        for d in descs:
            d.wait()
        return _

    lax.fori_loop(0, NBLK, body, None)
    plsc.subcore_barrier()
    pltpu.sync_copy(acc.at[pl.ds(s * STRIPE, STRIPE)],
                    out_hbm.at[pl.ds(c * HALF + s * STRIPE, STRIPE)])


def _sc_degree(dst_pad, ones16, zeros16):
    return pl.kernel(
        _degree_body,
        out_type=jax.ShapeDtypeStruct((NP, L), jnp.float32),
        mesh=_mesh(),
        compiler_params=_SC_PARAMS,
        scratch_types=[
            pltpu.VMEM((BLK,), jnp.int32),
            pltpu.VMEM((NCH, CH), jnp.int32),
            pltpu.VMEM((CH, L), jnp.float32),
            pltpu.VMEM_SHARED((ACC_ROWS, L), jnp.float32),
            pltpu.SemaphoreType.DMA,
        ],
    )(dst_pad, ones16, zeros16)


def _agg_body(u_hbm, src_hbm, dst_hbm, zeros_hbm, out_hbm, svm, dvm, lbuf,
              rvm0, rvm1, acc, semg, sems0, sems1):
    c = lax.axis_index("c")
    s = lax.axis_index("s")
    base = c * HALF
    rvms = (rvm0, rvm1)
    ssems = (sems0, sems1)
    _zero_acc(zeros_hbm, acc, s)
    plsc.subcore_barrier()

    def body(b, _):
        off = s * EPT + b * BLK
        pltpu.sync_copy(src_hbm.at[pl.ds(off, BLK)], svm)
        pltpu.sync_copy(dst_hbm.at[pl.ds(off, BLK)], dvm)
        _localize(dvm, lbuf, base)
        for j in range(NCH):
            p = j % 2
            if j >= 2:
                pltpu.make_async_copy(rvms[p], acc.at[lbuf.at[j - 2]],
                                      ssems[p]).wait()
            pltpu.async_copy(u_hbm.at[svm.at[pl.ds(j * CH, CH)]],
                             rvms[p], semg).wait()
            pltpu.async_copy(rvms[p], acc.at[lbuf.at[j]], ssems[p], add=True)
        for j in (NCH - 2, NCH - 1):
            p = j % 2
            pltpu.make_async_copy(rvms[p], acc.at[lbuf.at[j]],
                                  ssems[p]).wait()
        return _

    lax.fori_loop(0, NBLK, body, None)
    plsc.subcore_barrier()
    pltpu.sync_copy(acc.at[pl.ds(s * STRIPE, STRIPE)],
                    out_hbm.at[pl.ds(c * HALF + s * STRIPE, STRIPE)])


def _sc_aggregate(u_pad, src_pad, dst_pad, zeros72):
    return pl.kernel(
        _agg_body,
        out_type=jax.ShapeDtypeStruct((NP, FP), jnp.float32),
        mesh=_mesh(),
        compiler_params=_SC_PARAMS,
        scratch_types=[
            pltpu.VMEM((BLK,), jnp.int32),
            pltpu.VMEM((BLK,), jnp.int32),
            pltpu.VMEM((NCH, CH), jnp.int32),
            pltpu.VMEM((CH, FP), jnp.float32),
            pltpu.VMEM((CH, FP), jnp.float32),
            pltpu.VMEM_SHARED((ACC_ROWS, FP), jnp.float32),
            pltpu.SemaphoreType.DMA,
            pltpu.SemaphoreType.DMA,
            pltpu.SemaphoreType.DMA,
        ],
    )(u_pad, src_pad, dst_pad, zeros72)


def _pool_body(y_hbm, batch_hbm, neg_hbm, out_hbm, ybuf, bbuf, part):
    c = lax.axis_index("c")
    s = lax.axis_index("s")
    wid = s * NC + c
    rbase = wid * PTILE
    pltpu.sync_copy(neg_hbm, part)
    pltpu.sync_copy(batch_hbm.at[pl.ds(rbase, PTILE)], bbuf)
    iota = lax.iota(jnp.int32, L)
    zeros16 = jnp.zeros((L,), jnp.int32)
    ones_mask = zeros16 < 1

    bprev = plsc.load_gather(bbuf, [zeros16])
    runs = [jnp.full((L,), NEG, jnp.float32) for _ in POFF]

    carry = tuple([bprev] + runs)
    for sb in range(PTILE // PSUB):
        pltpu.sync_copy(y_hbm.at[pl.ds(rbase + sb * PSUB, PSUB)], ybuf)

        def body(i, car, sb=sb):
            bprev = car[0]
            runs = list(car[1:])
            gi = jnp.full((L,), sb * PSUB, jnp.int32) + i
            bi = plsc.load_gather(bbuf, [gi])
            m = bi != bprev
            row = jnp.full((L,), i, jnp.int32)
            for k, off in enumerate(POFF):
                plsc.store_scatter(part, [bprev * FP + off + iota],
                                   runs[k], mask=m)
                yv = plsc.load_gather(ybuf, [row, iota + off])
                rk = jnp.where(m, jnp.full((L,), NEG, jnp.float32), runs[k])
                runs[k] = jnp.maximum(rk, yv)
            return tuple([bi] + runs)

        carry = lax.fori_loop(0, PSUB, body, carry)
    bprev = carry[0]
    for k, off in enumerate(POFF):
        plsc.store_scatter(part, [bprev * FP + off + iota], carry[1 + k],
                           mask=ones_mask)
    pltpu.sync_copy(part, out_hbm.at[wid])


def _sc_pool(y_pad, batch_pad, neg_const):
    return pl.kernel(
        _pool_body,
        out_type=jax.ShapeDtypeStruct((NW, PSEG * FP), jnp.float32),
        mesh=_mesh(),
        compiler_params=_SC_PARAMS_NOLAYOUT,
        scratch_types=[
            pltpu.VMEM((PSUB, FP), jnp.float32),
            pltpu.VMEM((PTILE,), jnp.int32),
            pltpu.VMEM((PSEG * FP,), jnp.float32),
        ],
    )(y_pad, batch_pad, neg_const)


ROWS_BLK = 1024
NROWB = NP // ROWS_BLK


def _tc_u1_body(x_ref, w_ref, deg_ref, u_ref):
    dinv = lax.rsqrt(deg_ref[:, 0:1] + 1.0)
    u_ref[...] = jnp.dot(x_ref[...], w_ref[...],
                         preferred_element_type=jnp.float32) * dinv


def _tc_u1(x80, w80, deg16):
    return pl.pallas_call(
        _tc_u1_body,
        grid=(NROWB,),
        in_specs=[
            pl.BlockSpec((ROWS_BLK, FP), lambda i: (i, 0)),
            pl.BlockSpec((FP, FP), lambda i: (0, 0)),
            pl.BlockSpec((ROWS_BLK, L), lambda i: (i, 0)),
        ],
        out_specs=pl.BlockSpec((ROWS_BLK, FP), lambda i: (i, 0)),
        out_shape=jax.ShapeDtypeStruct((NP, FP), jnp.float32),
    )(x80, w80, deg16)


def _tc_u2_body(x_ref, u1_ref, s1_ref, deg_ref, b1_ref, w2_ref, u2_ref):
    dinv = lax.rsqrt(deg_ref[:, 0:1] + 1.0)
    a = jax.nn.relu(dinv * (s1_ref[...] + u1_ref[...]) + b1_ref[...])
    h = x_ref[...] + a
    u2_ref[...] = jnp.dot(h, w2_ref[...],
                          preferred_element_type=jnp.float32) * dinv


def _tc_u2(x80, u1, s1, deg16, b1, w80):
    return pl.pallas_call(
        _tc_u2_body,
        grid=(NROWB,),
        in_specs=[
            pl.BlockSpec((ROWS_BLK, FP), lambda i: (i, 0)),
            pl.BlockSpec((ROWS_BLK, FP), lambda i: (i, 0)),
            pl.BlockSpec((ROWS_BLK, FP), lambda i: (i, 0)),
            pl.BlockSpec((ROWS_BLK, L), lambda i: (i, 0)),
            pl.BlockSpec((1, FP), lambda i: (0, 0)),
            pl.BlockSpec((FP, FP), lambda i: (0, 0)),
        ],
        out_specs=pl.BlockSpec((ROWS_BLK, FP), lambda i: (i, 0)),
        out_shape=jax.ShapeDtypeStruct((NP, FP), jnp.float32),
    )(x80, u1, s1, deg16, b1, w80)


def _tc_y_body(u2_ref, s2_ref, deg_ref, b2_ref, y_ref):
    dinv = lax.rsqrt(deg_ref[:, 0:1] + 1.0)
    y_ref[...] = dinv * (s2_ref[...] + u2_ref[...]) + b2_ref[...]


def _tc_y(u2, s2, deg16, b2):
    return pl.pallas_call(
        _tc_y_body,
        grid=(NROWB,),
        in_specs=[
            pl.BlockSpec((ROWS_BLK, FP), lambda i: (i, 0)),
            pl.BlockSpec((ROWS_BLK, FP), lambda i: (i, 0)),
            pl.BlockSpec((ROWS_BLK, L), lambda i: (i, 0)),
            pl.BlockSpec((1, FP), lambda i: (0, 0)),
        ],
        out_specs=pl.BlockSpec((ROWS_BLK, FP), lambda i: (i, 0)),
        out_shape=jax.ShapeDtypeStruct((NP, FP), jnp.float32),
    )(u2, s2, deg16, b2)


FC = 144


def _tc_head_body(pw_ref, pm_ref, wwfc_ref, bwfc_ref, wmfc_ref, bmfc_ref,
                  w0m_ref, w0w_ref, b0_ref, w1_ref, b1_ref, wo_ref, bo_ref,
                  out_ref):
    p_w = jnp.max(pw_ref[...], axis=0)[:G, :]
    p_m = jnp.max(pm_ref[...], axis=0)[:G, :]
    xw = jax.nn.relu(jnp.dot(p_w, wwfc_ref[...],
                             preferred_element_type=jnp.float32)
                     + bwfc_ref[...])
    xm = jax.nn.relu(jnp.dot(p_m, wmfc_ref[...],
                             preferred_element_type=jnp.float32)
                     + bmfc_ref[...])
    z = jax.nn.relu(jnp.dot(xm, w0m_ref[...],
                            preferred_element_type=jnp.float32)
                    + jnp.dot(xw, w0w_ref[...],
                              preferred_element_type=jnp.float32)
                    + b0_ref[...])
    z = jax.nn.relu(jnp.dot(z, w1_ref[...],
                            preferred_element_type=jnp.float32) + b1_ref[...])
    out_ref[...] = jnp.dot(z, wo_ref[...],
                           preferred_element_type=jnp.float32) + bo_ref[...]


def _tc_head(pw3, pm3, wwfc, bwfc, wmfc, bmfc, w0m, w0w, b0, w1, b1, wo, bo):
    return pl.pallas_call(
        _tc_head_body,
        out_shape=jax.ShapeDtypeStruct((G, 8), jnp.float32),
    )(pw3, pm3, wwfc, bwfc, wmfc, bmfc, w0m, w0w, b0, w1, b1, wo, bo)


def _pad2(w, rows, cols):
    return jnp.pad(w, ((0, rows - w.shape[0]), (0, cols - w.shape[1])))


def _branch(x, edge_index, batch, W1, b1, W2, b2, consts):
    ones16, zeros16, zeros72, neg_const = consts
    x72 = jnp.pad(x, ((0, NP - N), (0, FP - F)))
    npad = EP - E
    pad_src = (jnp.arange(npad, dtype=jnp.int32) * 997) % N
    src_pad = jnp.concatenate([edge_index[0], pad_src])
    dst_pad = jnp.concatenate(
        [edge_index[1], jnp.full((npad,), 2 * NP, jnp.int32)])
    batch_pad = jnp.pad(batch, (0, NP - N), constant_values=G)

    w1p = _pad2(W1, FP, FP)
    w2p = _pad2(W2, FP, FP)
    b1p = jnp.pad(b1, (0, FP - F)).reshape(1, FP)
    b2p = jnp.pad(b2, (0, FP - F)).reshape(1, FP)

    deg16 = _sc_degree(dst_pad, ones16, zeros16)
    u1 = _tc_u1(x72, w1p, deg16)
    s1 = _sc_aggregate(u1, src_pad, dst_pad, zeros72)
    u2 = _tc_u2(x72, u1, s1, deg16, b1p, w2p)
    s2 = _sc_aggregate(u2, src_pad, dst_pad, zeros72)
    y = _tc_y(u2, s2, deg16, b2p)
    parts = _sc_pool(y, batch_pad, neg_const)
    return parts.reshape(NW, PSEG, FP)


def kernel(wild_x, wild_edge_index, wild_batch, mutant_x, mutant_edge_index,
           mutant_batch, Ww1, bw1, Ww2, bw2, Wwfc1, bwfc1, Wm1, bm1, Wm2, bm2,
           Wmfc1, bmfc1, mlp_W0, mlp_b0, mlp_W1, mlp_b1, mlp_Wo, mlp_bo):
    ones16 = jnp.ones((CH, L), jnp.float32)
    zeros16 = jnp.zeros((STRIPE, L), jnp.float32)
    zeros72 = jnp.zeros((STRIPE, FP), jnp.float32)
    neg_const = jnp.full((PSEG * FP,), NEG, jnp.float32)
    consts = (ones16, zeros16, zeros72, neg_const)

    pw = _branch(wild_x, wild_edge_index, wild_batch, Ww1, bw1, Ww2, bw2,
                 consts)
    pm = _branch(mutant_x, mutant_edge_index, mutant_batch, Wm1, bm1, Wm2,
                 bm2, consts)

    wwfc = _pad2(Wwfc1, FP, FC)
    wmfc = _pad2(Wmfc1, FP, FC)
    bwfc = jnp.pad(bwfc1, (0, FC - OUT_DIM)).reshape(1, FC)
    bmfc = jnp.pad(bmfc1, (0, FC - OUT_DIM)).reshape(1, FC)
    w0m = _pad2(mlp_W0[:OUT_DIM], FC, FP)
    w0w = _pad2(mlp_W0[OUT_DIM:], FC, FP)
    b0 = jnp.pad(mlp_b0, (0, FP - mlp_b0.shape[0])).reshape(1, FP)
    w1 = _pad2(mlp_W1, FP, FP)
    b1 = jnp.pad(mlp_b1, (0, FP - mlp_b1.shape[0])).reshape(1, FP)
    wo = _pad2(mlp_Wo, FP, 8)
    bo = jnp.pad(mlp_bo, (0, 8 - 1)).reshape(1, 8)

    out8 = _tc_head(pw, pm, wwfc, bwfc, wmfc, bmfc, w0m, w0w, b0, w1, b1,
                    wo, bo)
    return out8[:, 0:1]


# speedup vs baseline: 14.7169x; 1.2804x over previous
import functools

import jax
import jax.numpy as jnp
from jax import lax
from jax.experimental import pallas as pl
from jax.experimental.pallas import tpu as pltpu
from jax.experimental.pallas import tpu_sc as plsc

N = 50000
E = 800000
F = 71
OUT_DIM = 142
G = 128

FP = 72
NP = 50176
HALF = 25088
TRASH = HALF
ACC_ROWS = HALF + 8
NC, NS, L = 2, 16, 16
NW = NC * NS
CH = 96
NCH = 14
BLK = CH * NCH
NBLK = 38
EPT = BLK * NBLK
EP = EPT * NS
STRIPE = HALF // NS
PTILE = NP // NW
PSUB = 224
PSEG = 136
POFF = (0, 16, 32, 48, 56)
NEG = -3.0e38


def _mesh():
    return plsc.VectorSubcoreMesh(
        core_axis_name="c", subcore_axis_name="s", num_cores=NC,
        num_subcores=NS)


_SC_PARAMS = pltpu.CompilerParams(use_tc_tiling_on_sc=False)
_SC_PARAMS_NOLAYOUT = pltpu.CompilerParams(
    use_tc_tiling_on_sc=False, needs_layout_passes=False)


def _localize(dvm, lbuf, base):
    iota = lax.iota(jnp.int32, L)
    for k in range(BLK // L):
        dv = dvm[pl.ds(k * L, L)]
        local = dv - base
        m = (local >= 0) & (local < HALF)
        lidx = jnp.where(m, local, TRASH + (iota & 7))
        lbuf[k // (CH // L), pl.ds((k % (CH // L)) * L, L)] = lidx


def _zero_acc(zeros_hbm, acc, s):
    pltpu.sync_copy(zeros_hbm.at[pl.ds(0, STRIPE)],
                    acc.at[pl.ds(s * STRIPE, STRIPE)])
    @pl.when(s == NS - 1)
    def _():
        pltpu.sync_copy(zeros_hbm.at[pl.ds(0, ACC_ROWS - HALF)],
                        acc.at[pl.ds(HALF, ACC_ROWS - HALF)])


def _degree_body(dst_hbm, ones_hbm, zeros_hbm, out_hbm, dvm, lbuf, ovm, acc,
                 sem):
    c = lax.axis_index("c")
    s = lax.axis_index("s")
    base = c * HALF
    _zero_acc(zeros_hbm, acc, s)
    pltpu.sync_copy(ones_hbm, ovm)
    plsc.subcore_barrier()

    def body(b, _):
        off = s * EPT + b * BLK
        pltpu.sync_copy(dst_hbm.at[pl.ds(off, BLK)], dvm)
        _localize(dvm, lbuf, base)
        descs = [pltpu.async_copy(ovm, acc.at[lbuf.at[j]], sem, add=True)
                 for j in range(NCH)]
        for d in descs:
            d.wait()
        return _

    lax.fori_loop(0, NBLK, body, None)
    plsc.subcore_barrier()
    pltpu.sync_copy(acc.at[pl.ds(s * STRIPE, STRIPE)],
                    out_hbm.at[pl.ds(c * HALF + s * STRIPE, STRIPE)])


def _sc_degree(dst_pad, ones16, zeros16):
    return pl.kernel(
        _degree_body,
        out_type=jax.ShapeDtypeStruct((NP, L), jnp.float32),
        mesh=_mesh(),
        compiler_params=_SC_PARAMS,
        scratch_types=[
            pltpu.VMEM((BLK,), jnp.int32),
            pltpu.VMEM((NCH, CH), jnp.int32),
            pltpu.VMEM((CH, L), jnp.float32),
            pltpu.VMEM_SHARED((ACC_ROWS, L), jnp.float32),
            pltpu.SemaphoreType.DMA,
        ],
    )(dst_pad, ones16, zeros16)


def _agg_body(u_hbm, src_hbm, dst_hbm, zeros_hbm, out_hbm, svm, dvm, lbuf,
              rvm0, rvm1, acc, semg, sems0, sems1):
    c = lax.axis_index("c")
    s = lax.axis_index("s")
    base = c * HALF
    rvms = (rvm0, rvm1)
    ssems = (sems0, sems1)
    _zero_acc(zeros_hbm, acc, s)
    plsc.subcore_barrier()

    def body(b, _):
        off = s * EPT + b * BLK
        pltpu.sync_copy(src_hbm.at[pl.ds(off, BLK)], svm)
        pltpu.sync_copy(dst_hbm.at[pl.ds(off, BLK)], dvm)
        _localize(dvm, lbuf, base)
        for j in range(NCH):
            p = j % 2
            if j >= 2:
                pltpu.make_async_copy(rvms[p], acc.at[lbuf.at[j - 2]],
                                      ssems[p]).wait()
            pltpu.async_copy(u_hbm.at[svm.at[pl.ds(j * CH, CH)]],
                             rvms[p], semg).wait()
            pltpu.async_copy(rvms[p], acc.at[lbuf.at[j]], ssems[p], add=True)
        for j in (NCH - 2, NCH - 1):
            p = j % 2
            pltpu.make_async_copy(rvms[p], acc.at[lbuf.at[j]],
                                  ssems[p]).wait()
        return _

    lax.fori_loop(0, NBLK, body, None)
    plsc.subcore_barrier()
    pltpu.sync_copy(acc.at[pl.ds(s * STRIPE, STRIPE)],
                    out_hbm.at[pl.ds(c * HALF + s * STRIPE, STRIPE)])


def _sc_aggregate(u_pad, src_pad, dst_pad, zeros72):
    return pl.kernel(
        _agg_body,
        out_type=jax.ShapeDtypeStruct((NP, FP), jnp.float32),
        mesh=_mesh(),
        compiler_params=_SC_PARAMS,
        scratch_types=[
            pltpu.VMEM((BLK,), jnp.int32),
            pltpu.VMEM((BLK,), jnp.int32),
            pltpu.VMEM((NCH, CH), jnp.int32),
            pltpu.VMEM((CH, FP), jnp.float32),
            pltpu.VMEM((CH, FP), jnp.float32),
            pltpu.VMEM_SHARED((ACC_ROWS, FP), jnp.float32),
            pltpu.SemaphoreType.DMA,
            pltpu.SemaphoreType.DMA,
            pltpu.SemaphoreType.DMA,
        ],
    )(u_pad, src_pad, dst_pad, zeros72)


def _pool_body(y_hbm, batch_hbm, neg_hbm, out_hbm, ybuf, bbuf, part):
    c = lax.axis_index("c")
    s = lax.axis_index("s")
    wid = s * NC + c
    rbase = wid * PTILE
    pltpu.sync_copy(neg_hbm, part)
    pltpu.sync_copy(batch_hbm.at[pl.ds(rbase, PTILE)], bbuf)
    iota = lax.iota(jnp.int32, L)
    zeros16 = jnp.zeros((L,), jnp.int32)
    ones_mask = zeros16 < 1

    bprev = plsc.load_gather(bbuf, [zeros16])
    runs = [jnp.full((L,), NEG, jnp.float32) for _ in POFF]

    carry = tuple([bprev] + runs)
    for sb in range(PTILE // PSUB):
        pltpu.sync_copy(y_hbm.at[pl.ds(rbase + sb * PSUB, PSUB)], ybuf)

        def body(i, car, sb=sb):
            bprev = car[0]
            runs = list(car[1:])
            gi = jnp.full((L,), sb * PSUB, jnp.int32) + i
            bi = plsc.load_gather(bbuf, [gi])
            m = bi != bprev
            row = jnp.full((L,), i, jnp.int32)
            for k, off in enumerate(POFF):
                plsc.store_scatter(part, [bprev * FP + off + iota],
                                   runs[k], mask=m)
                yv = plsc.load_gather(ybuf, [row, iota + off])
                rk = jnp.where(m, jnp.full((L,), NEG, jnp.float32), runs[k])
                runs[k] = jnp.maximum(rk, yv)
            return tuple([bi] + runs)

        carry = lax.fori_loop(0, PSUB, body, carry)
    bprev = carry[0]
    for k, off in enumerate(POFF):
        plsc.store_scatter(part, [bprev * FP + off + iota], carry[1 + k],
                           mask=ones_mask)
    pltpu.sync_copy(part, out_hbm.at[wid])


def _sc_pool(y_pad, batch_pad, neg_const):
    return pl.kernel(
        _pool_body,
        out_type=jax.ShapeDtypeStruct((NW, PSEG * FP), jnp.float32),
        mesh=_mesh(),
        compiler_params=_SC_PARAMS_NOLAYOUT,
        scratch_types=[
            pltpu.VMEM((PSUB, FP), jnp.float32),
            pltpu.VMEM((PTILE,), jnp.int32),
            pltpu.VMEM((PSEG * FP,), jnp.float32),
        ],
    )(y_pad, batch_pad, neg_const)


ROWS_BLK = 1024
NROWB = NP // ROWS_BLK


def _tc_u1_body(x_ref, w_ref, deg_ref, u_ref):
    dinv = lax.rsqrt(deg_ref[:, 0:1] + 1.0)
    u_ref[...] = jnp.dot(x_ref[...], w_ref[...],
                         preferred_element_type=jnp.float32) * dinv


def _tc_u1(x80, w80, deg16):
    return pl.pallas_call(
        _tc_u1_body,
        grid=(NROWB,),
        in_specs=[
            pl.BlockSpec((ROWS_BLK, FP), lambda i: (i, 0)),
            pl.BlockSpec((FP, FP), lambda i: (0, 0)),
            pl.BlockSpec((ROWS_BLK, L), lambda i: (i, 0)),
        ],
        out_specs=pl.BlockSpec((ROWS_BLK, FP), lambda i: (i, 0)),
        out_shape=jax.ShapeDtypeStruct((NP, FP), jnp.float32),
    )(x80, w80, deg16)


def _tc_u2_body(x_ref, u1_ref, s1_ref, deg_ref, b1_ref, w2_ref, u2_ref):
    dinv = lax.rsqrt(deg_ref[:, 0:1] + 1.0)
    a = jax.nn.relu(dinv * (s1_ref[...] + u1_ref[...]) + b1_ref[...])
    h = x_ref[...] + a
    u2_ref[...] = jnp.dot(h, w2_ref[...],
                          preferred_element_type=jnp.float32) * dinv


def _tc_u2(x80, u1, s1, deg16, b1, w80):
    return pl.pallas_call(
        _tc_u2_body,
        grid=(NROWB,),
        in_specs=[
            pl.BlockSpec((ROWS_BLK, FP), lambda i: (i, 0)),
            pl.BlockSpec((ROWS_BLK, FP), lambda i: (i, 0)),
            pl.BlockSpec((ROWS_BLK, FP), lambda i: (i, 0)),
            pl.BlockSpec((ROWS_BLK, L), lambda i: (i, 0)),
            pl.BlockSpec((1, FP), lambda i: (0, 0)),
            pl.BlockSpec((FP, FP), lambda i: (0, 0)),
        ],
        out_specs=pl.BlockSpec((ROWS_BLK, FP), lambda i: (i, 0)),
        out_shape=jax.ShapeDtypeStruct((NP, FP), jnp.float32),
    )(x80, u1, s1, deg16, b1, w80)


def _tc_y_body(u2_ref, s2_ref, deg_ref, b2_ref, y_ref):
    dinv = lax.rsqrt(deg_ref[:, 0:1] + 1.0)
    y_ref[...] = dinv * (s2_ref[...] + u2_ref[...]) + b2_ref[...]


def _tc_y(u2, s2, deg16, b2):
    return pl.pallas_call(
        _tc_y_body,
        grid=(NROWB,),
        in_specs=[
            pl.BlockSpec((ROWS_BLK, FP), lambda i: (i, 0)),
            pl.BlockSpec((ROWS_BLK, FP), lambda i: (i, 0)),
            pl.BlockSpec((ROWS_BLK, L), lambda i: (i, 0)),
            pl.BlockSpec((1, FP), lambda i: (0, 0)),
        ],
        out_specs=pl.BlockSpec((ROWS_BLK, FP), lambda i: (i, 0)),
        out_shape=jax.ShapeDtypeStruct((NP, FP), jnp.float32),
    )(u2, s2, deg16, b2)


FC = 144


def _tc_head_body(pw_ref, pm_ref, wwfc_ref, bwfc_ref, wmfc_ref, bmfc_ref,
                  w0m_ref, w0w_ref, b0_ref, w1_ref, b1_ref, wo_ref, bo_ref,
                  out_ref):
    p_w = jnp.max(pw_ref[...], axis=0)[:G, :]
    p_m = jnp.max(pm_ref[...], axis=0)[:G, :]
    xw = jax.nn.relu(jnp.dot(p_w, wwfc_ref[...],
                             preferred_element_type=jnp.float32)
                     + bwfc_ref[...])
    xm = jax.nn.relu(jnp.dot(p_m, wmfc_ref[...],
                             preferred_element_type=jnp.float32)
                     + bmfc_ref[...])
    z = jax.nn.relu(jnp.dot(xm, w0m_ref[...],
                            preferred_element_type=jnp.float32)
                    + jnp.dot(xw, w0w_ref[...],
                              preferred_element_type=jnp.float32)
                    + b0_ref[...])
    z = jax.nn.relu(jnp.dot(z, w1_ref[...],
                            preferred_element_type=jnp.float32) + b1_ref[...])
    out_ref[...] = jnp.dot(z, wo_ref[...],
                           preferred_element_type=jnp.float32) + bo_ref[...]


def _tc_head(pw3, pm3, wwfc, bwfc, wmfc, bmfc, w0m, w0w, b0, w1, b1, wo, bo):
    return pl.pallas_call(
        _tc_head_body,
        out_shape=jax.ShapeDtypeStruct((G, 8), jnp.float32),
    )(pw3, pm3, wwfc, bwfc, wmfc, bmfc, w0m, w0w, b0, w1, b1, wo, bo)


def _pad2(w, rows, cols):
    return jnp.pad(w, ((0, rows - w.shape[0]), (0, cols - w.shape[1])))


def _branch(x, edge_index, batch, W1, b1, W2, b2, consts):
    ones16, zeros16, zeros72, neg_const = consts
    x72 = jnp.pad(x, ((0, NP - N), (0, FP - F)))
    npad = EP - E
    pad_src = (jnp.arange(npad, dtype=jnp.int32) * 997) % N
    src_pad = jnp.concatenate([edge_index[0], pad_src])
    dst_pad = jnp.concatenate(
        [edge_index[1], jnp.full((npad,), 2 * NP, jnp.int32)])
    batch_pad = jnp.pad(batch, (0, NP - N), constant_values=G)

    w1p = _pad2(W1, FP, FP)
    w2p = _pad2(W2, FP, FP)
    b1p = jnp.pad(b1, (0, FP - F)).reshape(1, FP)
    b2p = jnp.pad(b2, (0, FP - F)).reshape(1, FP)

    deg16 = _sc_degree(dst_pad, ones16, zeros16)
    u1 = _tc_u1(x72, w1p, deg16)
    s1 = _sc_aggregate(u1, src_pad, dst_pad, zeros72)
    u2 = _tc_u2(x72, u1, s1, deg16, b1p, w2p)
    s2 = _sc_aggregate(u2, src_pad, dst_pad, zeros72)
    y = _tc_y(u2, s2, deg16, b2p)
    parts = _sc_pool(y, batch_pad, neg_const)
    return parts.reshape(NW, PSEG, FP)


def kernel(wild_x, wild_edge_index, wild_batch, mutant_x, mutant_edge_index,
           mutant_batch, Ww1, bw1, Ww2, bw2, Wwfc1, bwfc1, Wm1, bm1, Wm2, bm2,
           Wmfc1, bmfc1, mlp_W0, mlp_b0, mlp_W1, mlp_b1, mlp_Wo, mlp_bo):
    ones16 = jnp.ones((CH, L), jnp.float32)
    zeros16 = jnp.zeros((STRIPE, L), jnp.float32)
    zeros72 = jnp.zeros((STRIPE, FP), jnp.float32)
    neg_const = jnp.full((PSEG * FP,), NEG, jnp.float32)
    consts = (ones16, zeros16, zeros72, neg_const)

    pw = _branch(wild_x, wild_edge_index, wild_batch, Ww1, bw1, Ww2, bw2,
                 consts)
    pm = _branch(mutant_x, mutant_edge_index, mutant_batch, Wm1, bm1, Wm2,
                 bm2, consts)

    wwfc = _pad2(Wwfc1, FP, FC)
    wmfc = _pad2(Wmfc1, FP, FC)
    bwfc = jnp.pad(bwfc1, (0, FC - OUT_DIM)).reshape(1, FC)
    bmfc = jnp.pad(bmfc1, (0, FC - OUT_DIM)).reshape(1, FC)
    w0m = _pad2(mlp_W0[:OUT_DIM], FC, FP)
    w0w = _pad2(mlp_W0[OUT_DIM:], FC, FP)
    b0 = jnp.pad(mlp_b0, (0, FP - mlp_b0.shape[0])).reshape(1, FP)
    w1 = _pad2(mlp_W1, FP, FP)
    b1 = jnp.pad(mlp_b1, (0, FP - mlp_b1.shape[0])).reshape(1, FP)
    wo = _pad2(mlp_Wo, FP, 8)
    bo = jnp.pad(mlp_bo, (0, 8 - 1)).reshape(1, 8)

    out8 = _tc_head(pw, pm, wwfc, bwfc, wmfc, bmfc, w0m, w0w, b0, w1, b1,
                    wo, bo)
    return out8[:, 0:1]
